# baseline (speedup 1.0000x reference)
import functools

import jax
import jax.numpy as jnp
from jax import lax
from jax.experimental import pallas as pl
from jax.experimental.pallas import tpu as pltpu
from jax.experimental.pallas import tpu_sc as plsc

NC = 2
NS = 16
NW = NC * NS

SP = 24


@functools.lru_cache(maxsize=None)
def _make_emb(N: int, S: int, D: int):
    npw = N // NW
    assert npw % 2 == 0
    mesh = plsc.VectorSubcoreMesh(core_axis_name="c", subcore_axis_name="s")

    @functools.partial(
        pl.kernel,
        mesh=mesh,
        out_type=jax.ShapeDtypeStruct((N, SP, D), jnp.float32),
        scratch_types=[
            pltpu.VMEM((npw * SP,), jnp.int32),
            pltpu.VMEM((SP, D), jnp.float32),
            pltpu.VMEM((SP, D), jnp.float32),
            pltpu.SemaphoreType.DMA,
            pltpu.SemaphoreType.DMA,
            pltpu.SemaphoreType.DMA,
            pltpu.SemaphoreType.DMA,
        ],
    )
    def emb(table_hbm, idx_hbm, out_hbm, idx_v, b0, b1, g0, g1, s0, s1):
        wid = lax.axis_index("s") * NC + lax.axis_index("c")
        base = wid * npw
        bufs, gsem, ssem = (b0, b1), (g0, g1), (s0, s1)
        pltpu.sync_copy(idx_hbm.at[pl.ds(base * SP, npw * SP)], idx_v)

        def idx(c):
            return idx_v.at[pl.ds(c * SP, SP)]

        def g_start(c, b):
            pltpu.async_copy(table_hbm.at[idx(c)], bufs[b], gsem[b])

        def g_wait(c, b):
            pltpu.make_async_copy(table_hbm.at[idx(c)], bufs[b],
                                  gsem[b]).wait()

        g_start(0, 0)
        g_start(1, 1)

        def body(p, carry):
            for h in range(2):
                c = 2 * p + h
                b = h
                g_wait(c, b)
                pltpu.async_copy(bufs[b], out_hbm.at[base + c], ssem[b])
                pltpu.make_async_copy(bufs[b], out_hbm.at[base + c],
                                      ssem[b]).wait()

                @pl.when(p < npw // 2 - 1)
                def _():
                    g_start(c + 2, b)

            return carry

        lax.fori_loop(0, npw // 2, body, 0)

    return emb


def kernel(x, table):
    N, S = x.shape
    D = table.shape[1]
    xi = x.astype(jnp.int32)
    xe = jnp.concatenate([xi, xi[:, : SP - S]], axis=1)
    h = N // 2
    emb = _make_emb(h, S, D)
    big0 = emb(table, xe[:h].reshape(-1))
    big1 = emb(table, xe[h:].reshape(-1))
    return jnp.concatenate([big0[:, :S, :], big1[:, :S, :]], axis=0)


# speedup vs baseline: 1.4625x; 1.4625x over previous
import functools

import jax
import jax.numpy as jnp
from jax import lax
from jax.experimental import pallas as pl
from jax.experimental.pallas import tpu as pltpu
from jax.experimental.pallas import tpu_sc as plsc

NC = 2
NS = 16
NW = NC * NS

SP = 24


@functools.lru_cache(maxsize=None)
def _make_emb(N: int, S: int, D: int):
    npw = N // NW
    assert npw % 2 == 0
    mesh = plsc.VectorSubcoreMesh(core_axis_name="c", subcore_axis_name="s")

    @functools.partial(
        pl.kernel,
        mesh=mesh,
        out_type=jax.ShapeDtypeStruct((N, SP, D), jnp.float32),
        scratch_types=[
            pltpu.VMEM((npw * SP,), jnp.int32),
            pltpu.VMEM((SP, D), jnp.float32),
            pltpu.VMEM((SP, D), jnp.float32),
            pltpu.SemaphoreType.DMA,
            pltpu.SemaphoreType.DMA,
            pltpu.SemaphoreType.DMA,
            pltpu.SemaphoreType.DMA,
        ],
    )
    def emb(table_hbm, idx_hbm, out_hbm, idx_v, b0, b1, g0, g1, s0, s1):
        wid = lax.axis_index("s") * NC + lax.axis_index("c")
        base = wid * npw
        bufs, gsem, ssem = (b0, b1), (g0, g1), (s0, s1)
        pltpu.sync_copy(idx_hbm.at[pl.ds(base * SP, npw * SP)], idx_v)

        def idx(c):
            return idx_v.at[pl.ds(c * SP, SP)]

        def g_start(c, b):
            pltpu.async_copy(table_hbm.at[idx(c)], bufs[b], gsem[b])

        def g_wait(c, b):
            pltpu.make_async_copy(table_hbm.at[idx(c)], bufs[b],
                                  gsem[b]).wait()

        g_start(0, 0)
        g_start(1, 1)

        def body(p, carry):
            for h in range(2):
                c = 2 * p + h
                b = h
                g_wait(c, b)
                pltpu.async_copy(bufs[b], out_hbm.at[base + c], ssem[b])
                pltpu.make_async_copy(bufs[b], out_hbm.at[base + c],
                                      ssem[b]).wait()

                @pl.when(p < npw // 2 - 1)
                def _():
                    g_start(c + 2, b)

            return carry

        lax.fori_loop(0, npw // 2, body, 0)

    return emb


def kernel(x, table):
    N, S = x.shape
    D = table.shape[1]
    xi = x.astype(jnp.int32)
    xe = jnp.concatenate([xi, xi[:, : SP - S]], axis=1).reshape(-1)
    big = _make_emb(N, S, D)(table, xe)
    return big[:, :S, :]
